# baseline (speedup 1.0000x reference)
import functools

import jax
import jax.numpy as jnp
from jax import lax
from jax.experimental import pallas as pl
from jax.experimental.pallas import tpu as pltpu
from jax.experimental.pallas import tpu_sc as plsc

_EPP = 2

_SC_E = 16
_NW = 32
_CH = 32768


def _sc_stream_body(wg_hbm, wu_hbm, wd_hbm, out_hbm, buf, sems):
    c = lax.axis_index("c")
    s = lax.axis_index("s")
    wid = s * 2 + c
    ew = 1024 * 512
    start = (64 - _SC_E) * ew
    per_w = _SC_E * ew // _NW
    nchunk = per_w // _CH
    handles = [None, None]
    k = 0
    for src in (wg_hbm, wu_hbm, wd_hbm):
        for j in range(nchunk):
            b = k % 2
            if handles[b] is not None:
                handles[b].wait()
            off = start + wid * per_w + j * _CH
            handles[b] = pltpu.async_copy(
                src.at[pl.ds(off, _CH)], buf.at[b], sems.at[b])
            k += 1
    for b in range(2):
        if handles[b] is not None:
            handles[b].wait()

    @pl.when(wid == 0)
    def _():
        pltpu.sync_copy(buf.at[0, pl.ds(0, 16)], out_hbm)


@functools.partial(
    pl.kernel,
    out_type=jax.ShapeDtypeStruct((16,), jnp.float32),
    mesh=plsc.VectorSubcoreMesh(core_axis_name="c", subcore_axis_name="s"),
    scratch_types=[
        pltpu.VMEM((2, _CH), jnp.float32),
        pltpu.SemaphoreType.DMA((2,)),
    ],
)
def _sc_stream(wg_hbm, wu_hbm, wd_hbm, out_hbm, buf, sems):
    _sc_stream_body(wg_hbm, wu_hbm, wd_hbm, out_hbm, buf, sems)


def _moe_body(x_ref, wr_ref, wg_ref, wu_ref, wd_ref, out_ref, w_scr):
    step = pl.program_id(0)
    n_experts = pl.num_programs(0) * _EPP

    @pl.when(step == 0)
    def _router():
        x = x_ref[...]
        logits = jax.lax.dot_general(
            x, wr_ref[...], (((1,), (1,)), ((), ())),
            preferred_element_type=jnp.float32)
        m = jnp.max(logits, axis=-1, keepdims=True)
        ex = jnp.exp(logits - m)
        scores = ex / jnp.sum(ex, axis=-1, keepdims=True)
        idx = jax.lax.broadcasted_iota(jnp.int32, scores.shape, 1)
        m1 = jnp.max(scores, axis=-1, keepdims=True)
        i1 = jnp.min(jnp.where(scores == m1, idx, n_experts), axis=-1,
                     keepdims=True)
        oh1 = idx == i1
        s2 = jnp.where(oh1, -1.0, scores)
        m2 = jnp.max(s2, axis=-1, keepdims=True)
        i2 = jnp.min(jnp.where(s2 == m2, idx, n_experts), axis=-1,
                     keepdims=True)
        oh2 = idx == i2
        w_scr[...] = jnp.where(oh1, m1, 0.0) + jnp.where(oh2, m2, 0.0)
        out_ref[...] = jnp.zeros_like(out_ref)

    x = x_ref[...]
    w = w_scr[...]
    lane = jax.lax.broadcasted_iota(jnp.int32, w.shape, 1)
    acc = jnp.zeros_like(out_ref)
    for j in range(_EPP):
        eid = step * _EPP + j
        h1 = jnp.dot(x, wg_ref[j], preferred_element_type=jnp.float32)
        h2 = jnp.dot(x, wu_ref[j], preferred_element_type=jnp.float32)
        g = h1 * (1.0 / (1.0 + jnp.exp(-h1)))
        y = jnp.dot(g * h2, wd_ref[j], preferred_element_type=jnp.float32)
        wcol = jnp.sum(jnp.where(lane == eid, w, 0.0), axis=1, keepdims=True)
        acc = acc + y * wcol
    out_ref[...] += acc


def kernel(x, Wr, Wg, Wu, Wd):
    b, s, h = x.shape
    e, _, f = Wg.shape
    n = b * s
    xf = x.reshape(n, h)
    out = pl.pallas_call(
        _moe_body,
        grid=(e // _EPP,),
        in_specs=[
            pl.BlockSpec((n, h), lambda i: (0, 0)),
            pl.BlockSpec((e, h), lambda i: (0, 0)),
            pl.BlockSpec((_EPP, h, f), lambda i: (i, 0, 0)),
            pl.BlockSpec((_EPP, h, f), lambda i: (i, 0, 0)),
            pl.BlockSpec((_EPP, f, h), lambda i: (i, 0, 0)),
        ],
        out_specs=pl.BlockSpec((n, h), lambda i: (0, 0)),
        out_shape=jax.ShapeDtypeStruct((n, h), jnp.float32),
        scratch_shapes=[pltpu.VMEM((n, e), jnp.float32)],
        compiler_params=pltpu.CompilerParams(
            dimension_semantics=("arbitrary",)),
    )(xf, Wr, Wg, Wu, Wd)
    dummy = _sc_stream(Wg.reshape(-1), Wu.reshape(-1), Wd.reshape(-1))
    return (out + 0.0 * dummy[0]).reshape(b, s, h)


# speedup vs baseline: 2.6447x; 2.6447x over previous
import functools

import jax
import jax.numpy as jnp
from jax import lax
from jax.experimental import pallas as pl
from jax.experimental.pallas import tpu as pltpu
from jax.experimental.pallas import tpu_sc as plsc

_EPP = 2

_SC_E = 16
_NW = 32
_CH = 32768


def _sc_stream_body(wg_hbm, wu_hbm, wd_hbm, out_hbm, buf_a, buf_b, sems):
    c = lax.axis_index("c")
    s = lax.axis_index("s")
    wid = s * 2 + c
    eo = 48 + wid // 2
    half = wid % 2
    handles = [None, None]
    k = 0
    for src in (wg_hbm, wu_hbm):
        for j in range(8):
            b = k % 2
            if handles[b] is not None:
                handles[b].wait()
            r0 = half * 512 + j * 64
            handles[b] = pltpu.async_copy(
                src.at[eo, pl.ds(r0, 64), :], buf_a.at[b], sems.at[b])
            k += 1
    for b in range(2):
        if handles[b] is not None:
            handles[b].wait()
    handles = [None, None]
    for j in range(16):
        b = k % 2
        if handles[b] is not None:
            handles[b].wait()
        r0 = half * 256 + j * 16
        handles[b] = pltpu.async_copy(
            wd_hbm.at[eo, pl.ds(r0, 16), :], buf_b.at[b], sems.at[b])
        k += 1
    for b in range(2):
        if handles[b] is not None:
            handles[b].wait()

    @pl.when(wid == 0)
    def _():
        pltpu.sync_copy(buf_a.at[0, 0, pl.ds(0, 16)], out_hbm)


@functools.partial(
    pl.kernel,
    out_type=jax.ShapeDtypeStruct((16,), jnp.float32),
    mesh=plsc.VectorSubcoreMesh(core_axis_name="c", subcore_axis_name="s"),
    scratch_types=[
        pltpu.VMEM((2, 64, 512), jnp.float32),
        pltpu.VMEM((2, 16, 1024), jnp.float32),
        pltpu.SemaphoreType.DMA((2,)),
    ],
)
def _sc_stream(wg_hbm, wu_hbm, wd_hbm, out_hbm, buf_a, buf_b, sems):
    _sc_stream_body(wg_hbm, wu_hbm, wd_hbm, out_hbm, buf_a, buf_b, sems)


def _moe_body(x_ref, wr_ref, wg_ref, wu_ref, wd_ref, out_ref, w_scr):
    step = pl.program_id(0)
    n_experts = pl.num_programs(0) * _EPP

    @pl.when(step == 0)
    def _router():
        x = x_ref[...]
        logits = jax.lax.dot_general(
            x, wr_ref[...], (((1,), (1,)), ((), ())),
            preferred_element_type=jnp.float32)
        m = jnp.max(logits, axis=-1, keepdims=True)
        ex = jnp.exp(logits - m)
        scores = ex / jnp.sum(ex, axis=-1, keepdims=True)
        idx = jax.lax.broadcasted_iota(jnp.int32, scores.shape, 1)
        m1 = jnp.max(scores, axis=-1, keepdims=True)
        i1 = jnp.min(jnp.where(scores == m1, idx, n_experts), axis=-1,
                     keepdims=True)
        oh1 = idx == i1
        s2 = jnp.where(oh1, -1.0, scores)
        m2 = jnp.max(s2, axis=-1, keepdims=True)
        i2 = jnp.min(jnp.where(s2 == m2, idx, n_experts), axis=-1,
                     keepdims=True)
        oh2 = idx == i2
        w_scr[...] = jnp.where(oh1, m1, 0.0) + jnp.where(oh2, m2, 0.0)
        out_ref[...] = jnp.zeros_like(out_ref)

    x = x_ref[...]
    w = w_scr[...]
    lane = jax.lax.broadcasted_iota(jnp.int32, w.shape, 1)
    acc = jnp.zeros_like(out_ref)
    for j in range(_EPP):
        eid = step * _EPP + j
        h1 = jnp.dot(x, wg_ref[j], preferred_element_type=jnp.float32)
        h2 = jnp.dot(x, wu_ref[j], preferred_element_type=jnp.float32)
        g = h1 * (1.0 / (1.0 + jnp.exp(-h1)))
        y = jnp.dot(g * h2, wd_ref[j], preferred_element_type=jnp.float32)
        wcol = jnp.sum(jnp.where(lane == eid, w, 0.0), axis=1, keepdims=True)
        acc = acc + y * wcol
    out_ref[...] += acc


def kernel(x, Wr, Wg, Wu, Wd):
    b, s, h = x.shape
    e, _, f = Wg.shape
    n = b * s
    xf = x.reshape(n, h)
    out = pl.pallas_call(
        _moe_body,
        grid=(e // _EPP,),
        in_specs=[
            pl.BlockSpec((n, h), lambda i: (0, 0)),
            pl.BlockSpec((e, h), lambda i: (0, 0)),
            pl.BlockSpec((_EPP, h, f), lambda i: (i, 0, 0)),
            pl.BlockSpec((_EPP, h, f), lambda i: (i, 0, 0)),
            pl.BlockSpec((_EPP, f, h), lambda i: (i, 0, 0)),
        ],
        out_specs=pl.BlockSpec((n, h), lambda i: (0, 0)),
        out_shape=jax.ShapeDtypeStruct((n, h), jnp.float32),
        scratch_shapes=[pltpu.VMEM((n, e), jnp.float32)],
        compiler_params=pltpu.CompilerParams(
            dimension_semantics=("arbitrary",)),
    )(xf, Wr, Wg, Wu, Wd)
    dummy = _sc_stream(Wg, Wu, Wd)
    return (out + 0.0 * dummy[0]).reshape(b, s, h)
